# baseline (device time: 14135 ns/iter reference)
import jax
import jax.numpy as jnp
from jax import lax
from jax.experimental import pallas as pl
from jax.experimental.pallas import tpu as pltpu

BS = 128
M = 1024
Q = M // 4
KQ = Q // BS
KY = 2 * KQ


def kernel(x):
    m, n = x.shape
    assert m == M
    out_dtype = jnp.bfloat16

    def body(x_ref, out_ref, sy, ry, sxf, rxf, szf, rzf):
        A = lax.axis_index("x")
        my_y = lax.axis_index("y")
        my_z = lax.axis_index("z")
        B = lax.rem(my_z, 2)
        zp = my_z + 1 - 2 * B
        ynbr = (A, 1 - my_y, my_z)
        xnbr = (1 - A, my_y, my_z)
        znbr = (A, my_y, zp)

        barrier_sem = pltpu.get_barrier_semaphore()
        for nbr in (ynbr, xnbr, znbr):
            pl.semaphore_signal(
                barrier_sem, inc=1, device_id=nbr,
                device_id_type=pl.DeviceIdType.MESH,
            )

        own = my_y * m
        rem = (1 - my_y) * m
        eq = (2 * A + B) * Q
        dq = 3 * Q - eq

        def yoff(k):
            return (eq + k * BS) if k < KQ else (dq + (k - KQ) * BS)

        def yrdma(k):
            r = yoff(k)
            return pltpu.make_async_remote_copy(
                src_ref=out_ref.at[pl.ds(own + r, BS), :],
                dst_ref=out_ref.at[pl.ds(own + r, BS), :],
                send_sem=sy.at[k],
                recv_sem=ry.at[k],
                device_id=ynbr,
                device_id_type=pl.DeviceIdType.MESH,
            )

        def fwd(k, ssem, rsem, dev):
            r = rem + eq + k * BS
            return pltpu.make_async_remote_copy(
                src_ref=out_ref.at[pl.ds(r, BS), :],
                dst_ref=out_ref.at[pl.ds(r, BS), :],
                send_sem=ssem.at[k],
                recv_sem=rsem.at[k],
                device_id=dev,
                device_id_type=pl.DeviceIdType.MESH,
            )

        out_ref[pl.ds(own, m), :] = x_ref[...].astype(out_dtype)

        pl.semaphore_wait(barrier_sem, 3)

        for k in range(KY):
            yrdma(k).start()

        for k in range(KQ):
            yrdma(k).wait_recv()
            fwd(k, sxf, rxf, xnbr).start()
            fwd(k, szf, rzf, znbr).start()

        for k in range(KQ, KY):
            yrdma(k).wait_recv()
        for k in range(KQ):
            fwd(k, sxf, rxf, xnbr).wait_recv()
            fwd(k, szf, rzf, znbr).wait_recv()
        for k in range(KY):
            yrdma(k).wait_send()
        for k in range(KQ):
            fwd(k, sxf, rxf, xnbr).wait_send()
            fwd(k, szf, rzf, znbr).wait_send()

    return pl.pallas_call(
        body,
        out_shape=jax.ShapeDtypeStruct((2 * m, n), out_dtype),
        in_specs=[pl.BlockSpec(memory_space=pltpu.VMEM)],
        out_specs=pl.BlockSpec(memory_space=pltpu.VMEM),
        scratch_shapes=[
            pltpu.SemaphoreType.DMA((KY,)),
            pltpu.SemaphoreType.DMA((KY,)),
            pltpu.SemaphoreType.DMA((KQ,)),
            pltpu.SemaphoreType.DMA((KQ,)),
            pltpu.SemaphoreType.DMA((KQ,)),
            pltpu.SemaphoreType.DMA((KQ,)),
        ],
        compiler_params=pltpu.CompilerParams(collective_id=0),
    )(x)


# device time: 13629 ns/iter; 1.0371x vs baseline; 1.0371x over previous
import jax
import jax.numpy as jnp
from jax import lax
from jax.experimental import pallas as pl
from jax.experimental.pallas import tpu as pltpu

BS = 64
M = 1024
Q = M // 4
KQ = Q // BS
KY = 2 * KQ


def kernel(x):
    m, n = x.shape
    assert m == M
    out_dtype = jnp.bfloat16

    def body(x_ref, out_ref, sy, ry, sxf, rxf, szf, rzf):
        A = lax.axis_index("x")
        my_y = lax.axis_index("y")
        my_z = lax.axis_index("z")
        B = lax.rem(my_z, 2)
        zp = my_z + 1 - 2 * B
        ynbr = (A, 1 - my_y, my_z)
        xnbr = (1 - A, my_y, my_z)
        znbr = (A, my_y, zp)

        barrier_sem = pltpu.get_barrier_semaphore()
        for nbr in (ynbr, xnbr, znbr):
            pl.semaphore_signal(
                barrier_sem, inc=1, device_id=nbr,
                device_id_type=pl.DeviceIdType.MESH,
            )

        own = my_y * m
        rem = (1 - my_y) * m
        eq = (2 * A + B) * Q
        dq = 3 * Q - eq

        def yoff(k):
            return (eq + k * BS) if k < KQ else (dq + (k - KQ) * BS)

        def yrdma(k):
            r = yoff(k)
            return pltpu.make_async_remote_copy(
                src_ref=out_ref.at[pl.ds(own + r, BS), :],
                dst_ref=out_ref.at[pl.ds(own + r, BS), :],
                send_sem=sy.at[k],
                recv_sem=ry.at[k],
                device_id=ynbr,
                device_id_type=pl.DeviceIdType.MESH,
            )

        def fwd(k, ssem, rsem, dev):
            r = rem + eq + k * BS
            return pltpu.make_async_remote_copy(
                src_ref=out_ref.at[pl.ds(r, BS), :],
                dst_ref=out_ref.at[pl.ds(r, BS), :],
                send_sem=ssem.at[k],
                recv_sem=rsem.at[k],
                device_id=dev,
                device_id_type=pl.DeviceIdType.MESH,
            )

        out_ref[pl.ds(own, m), :] = x_ref[...].astype(out_dtype)

        pl.semaphore_wait(barrier_sem, 3)

        for k in range(KY):
            yrdma(k).start()

        for k in range(KQ):
            yrdma(k).wait_recv()
            fwd(k, sxf, rxf, xnbr).start()
            fwd(k, szf, rzf, znbr).start()

        for k in range(KQ, KY):
            yrdma(k).wait_recv()
        for k in range(KQ):
            fwd(k, sxf, rxf, xnbr).wait_recv()
            fwd(k, szf, rzf, znbr).wait_recv()
        for k in range(KY):
            yrdma(k).wait_send()
        for k in range(KQ):
            fwd(k, sxf, rxf, xnbr).wait_send()
            fwd(k, szf, rzf, znbr).wait_send()

    return pl.pallas_call(
        body,
        out_shape=jax.ShapeDtypeStruct((2 * m, n), out_dtype),
        in_specs=[pl.BlockSpec(memory_space=pltpu.VMEM)],
        out_specs=pl.BlockSpec(memory_space=pltpu.VMEM),
        scratch_shapes=[
            pltpu.SemaphoreType.DMA((KY,)),
            pltpu.SemaphoreType.DMA((KY,)),
            pltpu.SemaphoreType.DMA((KQ,)),
            pltpu.SemaphoreType.DMA((KQ,)),
            pltpu.SemaphoreType.DMA((KQ,)),
            pltpu.SemaphoreType.DMA((KQ,)),
        ],
        compiler_params=pltpu.CompilerParams(collective_id=0),
    )(x)
